# baseline (speedup 1.0000x reference)
import functools
import jax
import jax.numpy as jnp
from jax import lax
from jax.experimental import pallas as pl
from jax.experimental.pallas import tpu as pltpu
from jax.experimental.pallas import tpu_sc as plsc

N = 50000
E = 800000
D = 64
Q = 4
QH = 16
T = 512
NC = 2
NS = 16
EP = 819200
ROWS = EP // 128
RPS = ROWS // NS
GR = 8
NCHUNK = RPS // GR
HPAIR = NCHUNK // 2
CE = GR * 128
NPAD = 50048
NPS = NPAD // NS
GT = 768
GPS = GT // NS

_f32 = jnp.float32
_sc_params = pltpu.CompilerParams(use_tc_tiling_on_sc=False)
_mesh = plsc.VectorSubcoreMesh(core_axis_name="c", subcore_axis_name="s",
                               num_cores=NC, num_subcores=NS)


def _seg_body(with_deg, src_hbm, dst_hbm, t0, t1, t2, t3, zq, zd1, ones1,
              m0_out, m1_out, m2_out, m3_out, *rest):
    if with_deg:
        (deg_out, sidx0, didx0, rows0, sidx1, didx1, rows1, ones_v,
         sem0, sem1, m_sh, deg_sh) = rest
    else:
        (sidx0, didx0, rows0, sidx1, didx1, rows1, ones_v,
         sem0, sem1, m_sh, deg_sh) = rest
    c = lax.axis_index("c")
    s = lax.axis_index("s")
    bufs = ((sidx0, didx0, rows0, sem0), (sidx1, didx1, rows1, sem1))

    def run(tab, m_out, do_deg):
        pltpu.sync_copy(zq.at[pl.ds(s * NPS, NPS), :],
                        m_sh.at[pl.ds(s * NPS, NPS), :])
        if do_deg:
            pltpu.sync_copy(zd1.at[pl.ds(s * NPS, NPS), :],
                            deg_sh.at[pl.ds(s * NPS, NPS), :])
            pltpu.sync_copy(ones1, ones_v)
        plsc.subcore_barrier()

        row0 = s * RPS

        def load_fire(r, bi):
            si, di, ro, se = bufs[bi]
            pltpu.sync_copy(src_hbm.at[pl.ds(r, GR), :], si)
            pltpu.sync_copy(dst_hbm.at[pl.ds(r, GR), :], di)
            for j in range(GR):
                pltpu.async_copy(tab.at[si.at[j]],
                                 ro.at[pl.ds(j * 128, 128), :], se)

        def drain(bi):
            _, _, ro, se = bufs[bi]
            pltpu.make_async_copy(tab.at[pl.ds(0, CE), :], ro, se).wait()

        def scatter(bi, do_deg):
            _, di, ro, _ = bufs[bi]
            for j in range(GR):
                pltpu.sync_copy(ro.at[pl.ds(j * 128, 128), :],
                                m_sh.at[di.at[j]], add=True)
                if do_deg:
                    pltpu.sync_copy(ones_v, deg_sh.at[di.at[j]], add=True)

        load_fire(row0, 0)

        def pair(t, carry):
            a = row0 + (2 * t) * GR
            load_fire(a + GR, 1)
            drain(0)
            scatter(0, do_deg)

            @pl.when(t < HPAIR - 1)
            def _():
                load_fire(a + 2 * GR, 0)
            drain(1)
            scatter(1, do_deg)
            return carry
        lax.fori_loop(0, HPAIR, pair, 0)

        plsc.subcore_barrier()
        pltpu.sync_copy(m_sh.at[pl.ds(s * NPS, NPS), :],
                        m_out.at[pl.ds(s * NPS, NPS), :])
        if do_deg:
            pltpu.sync_copy(deg_sh.at[pl.ds(s * NPS, NPS), :],
                            deg_out.at[pl.ds(s * NPS, NPS), :])

    @pl.when(c == 0)
    def _():
        run(t0, m0_out, with_deg)
        run(t1, m1_out, False)

    @pl.when(c == 1)
    def _():
        run(t2, m2_out, False)
        run(t3, m3_out, False)


def _make_seg(with_deg):
    out_type = [jax.ShapeDtypeStruct((NPAD, QH), _f32) for _ in range(Q)]
    if with_deg:
        out_type.append(jax.ShapeDtypeStruct((NPAD, 8), _f32))
    scratch = [
        pltpu.VMEM((GR, 128), jnp.int32),
        pltpu.VMEM((GR, 128), jnp.int32),
        pltpu.VMEM((CE, QH), _f32),
        pltpu.VMEM((GR, 128), jnp.int32),
        pltpu.VMEM((GR, 128), jnp.int32),
        pltpu.VMEM((CE, QH), _f32),
        pltpu.VMEM((128, 8), _f32),
        pltpu.SemaphoreType.DMA,
        pltpu.SemaphoreType.DMA,
        pltpu.VMEM_SHARED((NPAD, QH), _f32),
        pltpu.VMEM_SHARED((NPAD, 8), _f32),
    ]
    return pl.kernel(functools.partial(_seg_body, with_deg),
                     out_type=out_type, mesh=_mesh, scratch_types=scratch,
                     compiler_params=_sc_params)


def _gather_body(o0, o1, o2, o3, tidx, g0, g1, g2, g3, idx_v, rows_v, sem):
    c = lax.axis_index("c")
    s = lax.axis_index("s")

    def run(tab, g_out):
        pltpu.sync_copy(tidx.at[pl.ds(s * GPS, GPS)], idx_v)
        pltpu.async_copy(tab.at[idx_v], rows_v, sem).wait()
        pltpu.sync_copy(rows_v, g_out.at[pl.ds(s * GPS, GPS), :])

    @pl.when(c == 0)
    def _():
        run(o0, g0)
        run(o1, g1)

    @pl.when(c == 1)
    def _():
        run(o2, g2)
        run(o3, g3)


_gather = pl.kernel(
    _gather_body,
    out_type=[jax.ShapeDtypeStruct((GT, QH), _f32) for _ in range(Q)],
    mesh=_mesh,
    scratch_types=[pltpu.VMEM((GPS,), jnp.int32),
                   pltpu.VMEM((GPS, QH), _f32),
                   pltpu.SemaphoreType.DMA],
    compiler_params=_sc_params)


NB = 2000
GRID = N // NB

_hi = lax.Precision.HIGHEST


def _in_body(x_ref, w_ref, b_ref, *o_refs):
    x = x_ref[...]
    w = w_ref[...]
    z = (x[:, 0:1] * w[0:1, :] + x[:, 1:2] * w[1:2, :] + x[:, 2:3] * w[2:3, :]
         + b_ref[...])
    z = jnp.maximum(z, 0.0)
    for q in range(Q):
        o_refs[q][...] = z[:, q * QH:(q + 1) * QH]


def _input_layer(x, W_in, b_in):
    return pl.pallas_call(
        _in_body,
        grid=(GRID,),
        in_specs=[pl.BlockSpec((NB, 3), lambda i: (i, 0)),
                  pl.BlockSpec((3, D), lambda i: (0, 0)),
                  pl.BlockSpec((1, D), lambda i: (0, 0))],
        out_specs=[pl.BlockSpec((NB, QH), lambda i: (i, 0))] * Q,
        out_shape=[jax.ShapeDtypeStruct((N, QH), _f32)] * Q,
    )(x, W_in, b_in.reshape(1, D))


def _layer_body(final, h0, h1, h2, h3, m0, m1, m2, m3, deg_ref,
                ws_ref, wn_ref, b_ref, wo_ref, bo_ref, *o_refs):
    rdeg = 1.0 / jnp.maximum(deg_ref[:, 0:1], 1.0)
    ws = ws_ref[...]
    wn = wn_ref[...]
    hs = (h0, h1, h2, h3)
    ms = (m0, m1, m2, m3)
    z = b_ref[...]
    for q in range(Q):
        sl = slice(q * QH, (q + 1) * QH)
        z = z + jnp.dot(hs[q][...], ws[sl], preferred_element_type=_f32,
                        precision=_hi)
        z = z + jnp.dot(ms[q][...] * rdeg, wn[sl],
                        preferred_element_type=_f32, precision=_hi)
    z = jnp.maximum(z, 0.0)
    if final:
        z = jnp.dot(z, wo_ref[...], preferred_element_type=_f32,
                    precision=_hi) + bo_ref[...]
    for q in range(Q):
        o_refs[q][...] = z[:, q * QH:(q + 1) * QH]


def _dense_layer(final, hq, mq, deg, Ws, Wn, b, Wo, bo):
    return pl.pallas_call(
        functools.partial(_layer_body, final),
        grid=(GRID,),
        in_specs=[pl.BlockSpec((NB, QH), lambda i: (i, 0))] * Q
        + [pl.BlockSpec((NB, QH), lambda i: (i, 0))] * Q
        + [pl.BlockSpec((NB, 8), lambda i: (i, 0)),
           pl.BlockSpec((D, D), lambda i: (0, 0)),
           pl.BlockSpec((D, D), lambda i: (0, 0)),
           pl.BlockSpec((1, D), lambda i: (0, 0)),
           pl.BlockSpec((D, D), lambda i: (0, 0)),
           pl.BlockSpec((1, D), lambda i: (0, 0))],
        out_specs=[pl.BlockSpec((NB, QH), lambda i: (i, 0))] * Q,
        out_shape=[jax.ShapeDtypeStruct((N, QH), _f32)] * Q,
    )(*hq, *mq, deg, Ws, Wn, b.reshape(1, D), Wo, bo.reshape(1, D))


def _readout_body(g0, g1, g2, g3, fin_ref, pi_ref):
    gs = (g0, g1, g2, g3)
    score = jnp.zeros((T, 1), _f32)
    for q in range(Q):
        ta = gs[q][0:T, :]
        ag = gs[q][T:T + 1, :]
        score = score + jnp.sum(ta * ag, axis=1, keepdims=True)
    score = score * 0.125
    score = jnp.where(fin_ref[...] > 0, -jnp.inf, score)
    mx = jnp.max(score)
    e = jnp.exp(score - mx)
    pi_ref[...] = e / jnp.sum(e)


def _readout(gq, fin):
    return pl.pallas_call(
        _readout_body,
        in_specs=[pl.BlockSpec((GT, QH), lambda: (0, 0))] * Q
        + [pl.BlockSpec((T, 1), lambda: (0, 0))],
        out_specs=pl.BlockSpec((T, 1), lambda: (0, 0)),
        out_shape=jax.ShapeDtypeStruct((T, 1), _f32),
    )(*gq, fin)


_seg_deg = _make_seg(True)
_seg = _make_seg(False)


def kernel(x, edge_index, ag_node_idx, task_node_indices, finished_task,
           W_in, b_in, W_self, W_neigh, b_l, W_out, b_out):
    src = edge_index[0]
    dst = edge_index[1]
    src2 = jnp.concatenate([src, jnp.zeros((EP - E,), jnp.int32)]).reshape(ROWS, 128)
    dst2 = jnp.concatenate([dst, jnp.full((EP - E,), N, jnp.int32)]).reshape(ROWS, 128)
    zq = jnp.zeros((NPAD, QH), _f32)
    zd1 = jnp.zeros((NPAD, 8), _f32)
    ones1 = jnp.ones((128, 8), _f32)

    hq = _input_layer(x, W_in, b_in)
    *mq, deg = _seg_deg(src2, dst2, *hq, zq, zd1, ones1)
    hq = _dense_layer(False, hq, mq, deg, W_self[0], W_neigh[0], b_l[0],
                      W_out, b_out)
    mq = _seg(src2, dst2, *hq, zq, zd1, ones1)
    hq = _dense_layer(False, hq, mq, deg, W_self[1], W_neigh[1], b_l[1],
                      W_out, b_out)
    mq = _seg(src2, dst2, *hq, zq, zd1, ones1)
    oq = _dense_layer(True, hq, mq, deg, W_self[2], W_neigh[2], b_l[2],
                      W_out, b_out)

    tidx = jnp.concatenate([task_node_indices,
                            jnp.full((GT - T,), ag_node_idx, jnp.int32)])
    gq = _gather(*oq, tidx)
    fin = finished_task.astype(_f32).reshape(T, 1)
    return _readout(gq, fin)


# speedup vs baseline: 1.0269x; 1.0269x over previous
import functools
import jax
import jax.numpy as jnp
from jax import lax
from jax.experimental import pallas as pl
from jax.experimental.pallas import tpu as pltpu
from jax.experimental.pallas import tpu_sc as plsc

N = 50000
E = 800000
D = 64
Q = 4
QH = 16
T = 512
NC = 2
NS = 16
EP = 819200
ROWS = EP // 128
RPS = ROWS // NS
GR = 8
NCHUNK = RPS // GR
HPAIR = NCHUNK // 2
CE = GR * 128
NPAD = 50048
NPS = NPAD // NS
GT = 768
GPS = GT // NS

_f32 = jnp.float32
_sc_params = pltpu.CompilerParams(use_tc_tiling_on_sc=False)
_mesh = plsc.VectorSubcoreMesh(core_axis_name="c", subcore_axis_name="s",
                               num_cores=NC, num_subcores=NS)


def _seg_body(with_deg, src_hbm, dst_hbm, t0, t1, t2, t3, zq, zd1, ones1,
              m0_out, m1_out, m2_out, m3_out, *rest):
    if with_deg:
        (deg_out, sidx0, didx0, rows0, sidx1, didx1, rows1, ones_v,
         sem0, sem1, ssc0, ssc1, sd0, sd1, m_sh, deg_sh) = rest
    else:
        (sidx0, didx0, rows0, sidx1, didx1, rows1, ones_v,
         sem0, sem1, ssc0, ssc1, sd0, sd1, m_sh, deg_sh) = rest
    c = lax.axis_index("c")
    s = lax.axis_index("s")
    bufs = ((sidx0, didx0, rows0, sem0, ssc0, sd0),
            (sidx1, didx1, rows1, sem1, ssc1, sd1))

    def run(tab, m_out, do_deg):
        pltpu.sync_copy(zq.at[pl.ds(s * NPS, NPS), :],
                        m_sh.at[pl.ds(s * NPS, NPS), :])
        if do_deg:
            pltpu.sync_copy(zd1.at[pl.ds(s * NPS, NPS), :],
                            deg_sh.at[pl.ds(s * NPS, NPS), :])
            pltpu.sync_copy(ones1, ones_v)
        plsc.subcore_barrier()

        row0 = s * RPS

        def load_fire(r, bi):
            si, di, ro, se, _, _ = bufs[bi]
            pltpu.sync_copy(src_hbm.at[pl.ds(r, GR), :], si)
            pltpu.sync_copy(dst_hbm.at[pl.ds(r, GR), :], di)
            for j in range(GR):
                pltpu.async_copy(tab.at[si.at[j]],
                                 ro.at[pl.ds(j * 128, 128), :], se)

        def drain_g(bi):
            _, _, ro, se, _, _ = bufs[bi]
            pltpu.make_async_copy(tab.at[pl.ds(0, CE), :], ro, se).wait()

        def fire_sc(bi, do_deg):
            _, di, ro, _, ssc, sd = bufs[bi]
            for j in range(GR):
                pltpu.async_copy(ro.at[pl.ds(j * 128, 128), :],
                                 m_sh.at[di.at[j]], ssc, add=True)
                if do_deg:
                    pltpu.async_copy(ones_v, deg_sh.at[di.at[j]], sd,
                                     add=True)

        def drain_sc(bi, do_deg):
            _, _, ro, _, ssc, sd = bufs[bi]
            pltpu.make_async_copy(tab.at[pl.ds(0, CE), :], ro, ssc).wait()
            if do_deg:
                for j in range(GR):
                    pltpu.make_async_copy(zd1.at[pl.ds(0, 128), :], ones_v,
                                          sd).wait()

        load_fire(row0, 0)

        def pair(t, carry):
            a = row0 + (2 * t) * GR

            @pl.when(t > 0)
            def _():
                drain_sc(1, do_deg)
            load_fire(a + GR, 1)
            drain_g(0)
            fire_sc(0, do_deg)

            @pl.when(t < HPAIR - 1)
            def _():
                drain_sc(0, do_deg)
                load_fire(a + 2 * GR, 0)
            drain_g(1)
            fire_sc(1, do_deg)
            return carry
        lax.fori_loop(0, HPAIR, pair, 0)
        drain_sc(0, do_deg)
        drain_sc(1, do_deg)

        plsc.subcore_barrier()
        pltpu.sync_copy(m_sh.at[pl.ds(s * NPS, NPS), :],
                        m_out.at[pl.ds(s * NPS, NPS), :])
        if do_deg:
            pltpu.sync_copy(deg_sh.at[pl.ds(s * NPS, NPS), :],
                            deg_out.at[pl.ds(s * NPS, NPS), :])

    @pl.when(c == 0)
    def _():
        run(t0, m0_out, with_deg)
        run(t1, m1_out, False)

    @pl.when(c == 1)
    def _():
        run(t2, m2_out, False)
        run(t3, m3_out, False)


def _make_seg(with_deg):
    out_type = [jax.ShapeDtypeStruct((NPAD, QH), _f32) for _ in range(Q)]
    if with_deg:
        out_type.append(jax.ShapeDtypeStruct((NPAD, 8), _f32))
    scratch = [
        pltpu.VMEM((GR, 128), jnp.int32),
        pltpu.VMEM((GR, 128), jnp.int32),
        pltpu.VMEM((CE, QH), _f32),
        pltpu.VMEM((GR, 128), jnp.int32),
        pltpu.VMEM((GR, 128), jnp.int32),
        pltpu.VMEM((CE, QH), _f32),
        pltpu.VMEM((128, 8), _f32),
        pltpu.SemaphoreType.DMA,
        pltpu.SemaphoreType.DMA,
        pltpu.SemaphoreType.DMA,
        pltpu.SemaphoreType.DMA,
        pltpu.SemaphoreType.DMA,
        pltpu.SemaphoreType.DMA,
        pltpu.VMEM_SHARED((NPAD, QH), _f32),
        pltpu.VMEM_SHARED((NPAD, 8), _f32),
    ]
    return pl.kernel(functools.partial(_seg_body, with_deg),
                     out_type=out_type, mesh=_mesh, scratch_types=scratch,
                     compiler_params=_sc_params)


def _gather_body(o0, o1, o2, o3, tidx, g0, g1, g2, g3, idx_v, rows_v, sem):
    c = lax.axis_index("c")
    s = lax.axis_index("s")

    def run(tab, g_out):
        pltpu.sync_copy(tidx.at[pl.ds(s * GPS, GPS)], idx_v)
        pltpu.async_copy(tab.at[idx_v], rows_v, sem).wait()
        pltpu.sync_copy(rows_v, g_out.at[pl.ds(s * GPS, GPS), :])

    @pl.when(c == 0)
    def _():
        run(o0, g0)
        run(o1, g1)

    @pl.when(c == 1)
    def _():
        run(o2, g2)
        run(o3, g3)


_gather = pl.kernel(
    _gather_body,
    out_type=[jax.ShapeDtypeStruct((GT, QH), _f32) for _ in range(Q)],
    mesh=_mesh,
    scratch_types=[pltpu.VMEM((GPS,), jnp.int32),
                   pltpu.VMEM((GPS, QH), _f32),
                   pltpu.SemaphoreType.DMA],
    compiler_params=_sc_params)


NB = 2000
GRID = N // NB

_hi = lax.Precision.HIGHEST


def _in_body(x_ref, w_ref, b_ref, *o_refs):
    x = x_ref[...]
    w = w_ref[...]
    z = (x[:, 0:1] * w[0:1, :] + x[:, 1:2] * w[1:2, :] + x[:, 2:3] * w[2:3, :]
         + b_ref[...])
    z = jnp.maximum(z, 0.0)
    for q in range(Q):
        o_refs[q][...] = z[:, q * QH:(q + 1) * QH]


def _input_layer(x, W_in, b_in):
    return pl.pallas_call(
        _in_body,
        grid=(GRID,),
        in_specs=[pl.BlockSpec((NB, 3), lambda i: (i, 0)),
                  pl.BlockSpec((3, D), lambda i: (0, 0)),
                  pl.BlockSpec((1, D), lambda i: (0, 0))],
        out_specs=[pl.BlockSpec((NB, QH), lambda i: (i, 0))] * Q,
        out_shape=[jax.ShapeDtypeStruct((N, QH), _f32)] * Q,
    )(x, W_in, b_in.reshape(1, D))


def _layer_body(final, h0, h1, h2, h3, m0, m1, m2, m3, deg_ref,
                ws_ref, wn_ref, b_ref, wo_ref, bo_ref, *o_refs):
    rdeg = 1.0 / jnp.maximum(deg_ref[:, 0:1], 1.0)
    ws = ws_ref[...]
    wn = wn_ref[...]
    hs = (h0, h1, h2, h3)
    ms = (m0, m1, m2, m3)
    z = b_ref[...]
    for q in range(Q):
        sl = slice(q * QH, (q + 1) * QH)
        z = z + jnp.dot(hs[q][...], ws[sl], preferred_element_type=_f32,
                        precision=_hi)
        z = z + jnp.dot(ms[q][...] * rdeg, wn[sl],
                        preferred_element_type=_f32, precision=_hi)
    z = jnp.maximum(z, 0.0)
    if final:
        z = jnp.dot(z, wo_ref[...], preferred_element_type=_f32,
                    precision=_hi) + bo_ref[...]
    for q in range(Q):
        o_refs[q][...] = z[:, q * QH:(q + 1) * QH]


def _dense_layer(final, hq, mq, deg, Ws, Wn, b, Wo, bo):
    return pl.pallas_call(
        functools.partial(_layer_body, final),
        grid=(GRID,),
        in_specs=[pl.BlockSpec((NB, QH), lambda i: (i, 0))] * Q
        + [pl.BlockSpec((NB, QH), lambda i: (i, 0))] * Q
        + [pl.BlockSpec((NB, 8), lambda i: (i, 0)),
           pl.BlockSpec((D, D), lambda i: (0, 0)),
           pl.BlockSpec((D, D), lambda i: (0, 0)),
           pl.BlockSpec((1, D), lambda i: (0, 0)),
           pl.BlockSpec((D, D), lambda i: (0, 0)),
           pl.BlockSpec((1, D), lambda i: (0, 0))],
        out_specs=[pl.BlockSpec((NB, QH), lambda i: (i, 0))] * Q,
        out_shape=[jax.ShapeDtypeStruct((N, QH), _f32)] * Q,
    )(*hq, *mq, deg, Ws, Wn, b.reshape(1, D), Wo, bo.reshape(1, D))


def _readout_body(g0, g1, g2, g3, fin_ref, pi_ref):
    gs = (g0, g1, g2, g3)
    score = jnp.zeros((T, 1), _f32)
    for q in range(Q):
        ta = gs[q][0:T, :]
        ag = gs[q][T:T + 1, :]
        score = score + jnp.sum(ta * ag, axis=1, keepdims=True)
    score = score * 0.125
    score = jnp.where(fin_ref[...] > 0, -jnp.inf, score)
    mx = jnp.max(score)
    e = jnp.exp(score - mx)
    pi_ref[...] = e / jnp.sum(e)


def _readout(gq, fin):
    return pl.pallas_call(
        _readout_body,
        in_specs=[pl.BlockSpec((GT, QH), lambda: (0, 0))] * Q
        + [pl.BlockSpec((T, 1), lambda: (0, 0))],
        out_specs=pl.BlockSpec((T, 1), lambda: (0, 0)),
        out_shape=jax.ShapeDtypeStruct((T, 1), _f32),
    )(*gq, fin)


_seg_deg = _make_seg(True)
_seg = _make_seg(False)


def kernel(x, edge_index, ag_node_idx, task_node_indices, finished_task,
           W_in, b_in, W_self, W_neigh, b_l, W_out, b_out):
    src = edge_index[0]
    dst = edge_index[1]
    src2 = jnp.concatenate([src, jnp.zeros((EP - E,), jnp.int32)]).reshape(ROWS, 128)
    dst2 = jnp.concatenate([dst, jnp.full((EP - E,), N, jnp.int32)]).reshape(ROWS, 128)
    zq = jnp.zeros((NPAD, QH), _f32)
    zd1 = jnp.zeros((NPAD, 8), _f32)
    ones1 = jnp.ones((128, 8), _f32)

    hq = _input_layer(x, W_in, b_in)
    *mq, deg = _seg_deg(src2, dst2, *hq, zq, zd1, ones1)
    hq = _dense_layer(False, hq, mq, deg, W_self[0], W_neigh[0], b_l[0],
                      W_out, b_out)
    mq = _seg(src2, dst2, *hq, zq, zd1, ones1)
    hq = _dense_layer(False, hq, mq, deg, W_self[1], W_neigh[1], b_l[1],
                      W_out, b_out)
    mq = _seg(src2, dst2, *hq, zq, zd1, ones1)
    oq = _dense_layer(True, hq, mq, deg, W_self[2], W_neigh[2], b_l[2],
                      W_out, b_out)

    tidx = jnp.concatenate([task_node_indices,
                            jnp.full((GT - T,), ag_node_idx, jnp.int32)])
    gq = _gather(*oq, tidx)
    fin = finished_task.astype(_f32).reshape(T, 1)
    return _readout(gq, fin)


# speedup vs baseline: 1.0599x; 1.0321x over previous
import functools
import jax
import jax.numpy as jnp
from jax import lax
from jax.experimental import pallas as pl
from jax.experimental.pallas import tpu as pltpu
from jax.experimental.pallas import tpu_sc as plsc

N = 50000
E = 800000
D = 64
Q = 4
QH = 16
T = 512
NC = 2
NS = 16
EP = 802816
ROWS = EP // 128
RPS = ROWS // NS
GR = 8
NCHUNK = RPS // GR
CE = GR * 128
NPAD = 50048
NPS = NPAD // NS
GT = 768
GPS = GT // NS

_f32 = jnp.float32
_sc_params = pltpu.CompilerParams(use_tc_tiling_on_sc=False)
_mesh = plsc.VectorSubcoreMesh(core_axis_name="c", subcore_axis_name="s",
                               num_cores=NC, num_subcores=NS)


def _seg_body(with_deg, src_hbm, dst_hbm, t0, t1, t2, t3, zq, zd1, ones1,
              m0_out, m1_out, m2_out, m3_out, *rest):
    if with_deg:
        deg_out, sidx, didx, rows, ones_v, sem, m_sh, deg_sh = rest
    else:
        sidx, didx, rows, ones_v, sem, m_sh, deg_sh = rest
    c = lax.axis_index("c")
    s = lax.axis_index("s")

    def run(tab, m_out, do_deg):
        pltpu.sync_copy(zq.at[pl.ds(s * NPS, NPS), :],
                        m_sh.at[pl.ds(s * NPS, NPS), :])
        if do_deg:
            pltpu.sync_copy(zd1.at[pl.ds(s * NPS, NPS), :],
                            deg_sh.at[pl.ds(s * NPS, NPS), :])
            pltpu.sync_copy(ones1, ones_v)
        plsc.subcore_barrier()

        row0 = s * RPS

        def chunk(k, carry):
            r = row0 + k * GR
            pltpu.sync_copy(src_hbm.at[pl.ds(r, GR), :], sidx)
            pltpu.sync_copy(dst_hbm.at[pl.ds(r, GR), :], didx)

            for j in range(GR):
                pltpu.async_copy(tab.at[sidx.at[j]],
                                 rows.at[pl.ds(j * 128, 128), :], sem)
            pltpu.make_async_copy(tab.at[pl.ds(0, CE), :], rows, sem).wait()
            for j in range(GR):
                pltpu.sync_copy(rows.at[pl.ds(j * 128, 128), :],
                                m_sh.at[didx.at[j]], add=True)
                if do_deg:
                    pltpu.sync_copy(ones_v, deg_sh.at[didx.at[j]], add=True)
            return carry
        lax.fori_loop(0, NCHUNK, chunk, 0)

        plsc.subcore_barrier()
        pltpu.sync_copy(m_sh.at[pl.ds(s * NPS, NPS), :],
                        m_out.at[pl.ds(s * NPS, NPS), :])
        if do_deg:
            pltpu.sync_copy(deg_sh.at[pl.ds(s * NPS, NPS), :],
                            deg_out.at[pl.ds(s * NPS, NPS), :])

    @pl.when(c == 0)
    def _():
        run(t0, m0_out, with_deg)
        run(t1, m1_out, False)

    @pl.when(c == 1)
    def _():
        run(t2, m2_out, False)
        run(t3, m3_out, False)


def _make_seg(with_deg):
    out_type = [jax.ShapeDtypeStruct((NPAD, QH), _f32) for _ in range(Q)]
    if with_deg:
        out_type.append(jax.ShapeDtypeStruct((NPAD, 8), _f32))
    scratch = [
        pltpu.VMEM((GR, 128), jnp.int32),
        pltpu.VMEM((GR, 128), jnp.int32),
        pltpu.VMEM((CE, QH), _f32),
        pltpu.VMEM((128, 8), _f32),
        pltpu.SemaphoreType.DMA,
        pltpu.VMEM_SHARED((NPAD, QH), _f32),
        pltpu.VMEM_SHARED((NPAD, 8), _f32),
    ]
    return pl.kernel(functools.partial(_seg_body, with_deg),
                     out_type=out_type, mesh=_mesh, scratch_types=scratch,
                     compiler_params=_sc_params)


def _gather_body(o0, o1, o2, o3, tidx, g0, g1, g2, g3, idx_v, rows_v, sem):
    c = lax.axis_index("c")
    s = lax.axis_index("s")

    def run(tab, g_out):
        pltpu.sync_copy(tidx.at[pl.ds(s * GPS, GPS)], idx_v)
        pltpu.async_copy(tab.at[idx_v], rows_v, sem).wait()
        pltpu.sync_copy(rows_v, g_out.at[pl.ds(s * GPS, GPS), :])

    @pl.when(c == 0)
    def _():
        run(o0, g0)
        run(o1, g1)

    @pl.when(c == 1)
    def _():
        run(o2, g2)
        run(o3, g3)


_gather = pl.kernel(
    _gather_body,
    out_type=[jax.ShapeDtypeStruct((GT, QH), _f32) for _ in range(Q)],
    mesh=_mesh,
    scratch_types=[pltpu.VMEM((GPS,), jnp.int32),
                   pltpu.VMEM((GPS, QH), _f32),
                   pltpu.SemaphoreType.DMA],
    compiler_params=_sc_params)


NB = 2000
GRID = N // NB

_hi = lax.Precision.HIGHEST


def _in_body(x_ref, w_ref, b_ref, *o_refs):
    x = x_ref[...]
    w = w_ref[...]
    z = (x[:, 0:1] * w[0:1, :] + x[:, 1:2] * w[1:2, :] + x[:, 2:3] * w[2:3, :]
         + b_ref[...])
    z = jnp.maximum(z, 0.0)
    for q in range(Q):
        o_refs[q][...] = z[:, q * QH:(q + 1) * QH]


def _input_layer(x, W_in, b_in):
    return pl.pallas_call(
        _in_body,
        grid=(GRID,),
        in_specs=[pl.BlockSpec((NB, 3), lambda i: (i, 0)),
                  pl.BlockSpec((3, D), lambda i: (0, 0)),
                  pl.BlockSpec((1, D), lambda i: (0, 0))],
        out_specs=[pl.BlockSpec((NB, QH), lambda i: (i, 0))] * Q,
        out_shape=[jax.ShapeDtypeStruct((N, QH), _f32)] * Q,
    )(x, W_in, b_in.reshape(1, D))


def _layer_body(final, h0, h1, h2, h3, m0, m1, m2, m3, deg_ref,
                ws_ref, wn_ref, b_ref, wo_ref, bo_ref, *o_refs):
    rdeg = 1.0 / jnp.maximum(deg_ref[:, 0:1], 1.0)
    ws = ws_ref[...]
    wn = wn_ref[...]
    hs = (h0, h1, h2, h3)
    ms = (m0, m1, m2, m3)
    z = b_ref[...]
    for q in range(Q):
        sl = slice(q * QH, (q + 1) * QH)
        z = z + jnp.dot(hs[q][...], ws[sl], preferred_element_type=_f32,
                        precision=_hi)
        z = z + jnp.dot(ms[q][...] * rdeg, wn[sl],
                        preferred_element_type=_f32, precision=_hi)
    z = jnp.maximum(z, 0.0)
    if final:
        z = jnp.dot(z, wo_ref[...], preferred_element_type=_f32,
                    precision=_hi) + bo_ref[...]
    for q in range(Q):
        o_refs[q][...] = z[:, q * QH:(q + 1) * QH]


def _dense_layer(final, hq, mq, deg, Ws, Wn, b, Wo, bo):
    return pl.pallas_call(
        functools.partial(_layer_body, final),
        grid=(GRID,),
        in_specs=[pl.BlockSpec((NB, QH), lambda i: (i, 0))] * Q
        + [pl.BlockSpec((NB, QH), lambda i: (i, 0))] * Q
        + [pl.BlockSpec((NB, 8), lambda i: (i, 0)),
           pl.BlockSpec((D, D), lambda i: (0, 0)),
           pl.BlockSpec((D, D), lambda i: (0, 0)),
           pl.BlockSpec((1, D), lambda i: (0, 0)),
           pl.BlockSpec((D, D), lambda i: (0, 0)),
           pl.BlockSpec((1, D), lambda i: (0, 0))],
        out_specs=[pl.BlockSpec((NB, QH), lambda i: (i, 0))] * Q,
        out_shape=[jax.ShapeDtypeStruct((N, QH), _f32)] * Q,
    )(*hq, *mq, deg, Ws, Wn, b.reshape(1, D), Wo, bo.reshape(1, D))


def _readout_body(g0, g1, g2, g3, fin_ref, pi_ref):
    gs = (g0, g1, g2, g3)
    score = jnp.zeros((T, 1), _f32)
    for q in range(Q):
        ta = gs[q][0:T, :]
        ag = gs[q][T:T + 1, :]
        score = score + jnp.sum(ta * ag, axis=1, keepdims=True)
    score = score * 0.125
    score = jnp.where(fin_ref[...] > 0, -jnp.inf, score)
    mx = jnp.max(score)
    e = jnp.exp(score - mx)
    pi_ref[...] = e / jnp.sum(e)


def _readout(gq, fin):
    return pl.pallas_call(
        _readout_body,
        in_specs=[pl.BlockSpec((GT, QH), lambda: (0, 0))] * Q
        + [pl.BlockSpec((T, 1), lambda: (0, 0))],
        out_specs=pl.BlockSpec((T, 1), lambda: (0, 0)),
        out_shape=jax.ShapeDtypeStruct((T, 1), _f32),
    )(*gq, fin)


_seg_deg = _make_seg(True)
_seg = _make_seg(False)


def kernel(x, edge_index, ag_node_idx, task_node_indices, finished_task,
           W_in, b_in, W_self, W_neigh, b_l, W_out, b_out):
    src = edge_index[0]
    dst = edge_index[1]
    src2 = jnp.concatenate([src, jnp.zeros((EP - E,), jnp.int32)]).reshape(ROWS, 128)
    dst2 = jnp.concatenate([dst, jnp.full((EP - E,), N, jnp.int32)]).reshape(ROWS, 128)
    zq = jnp.zeros((NPAD, QH), _f32)
    zd1 = jnp.zeros((NPAD, 8), _f32)
    ones1 = jnp.ones((128, 8), _f32)

    hq = _input_layer(x, W_in, b_in)
    *mq, deg = _seg_deg(src2, dst2, *hq, zq, zd1, ones1)
    hq = _dense_layer(False, hq, mq, deg, W_self[0], W_neigh[0], b_l[0],
                      W_out, b_out)
    mq = _seg(src2, dst2, *hq, zq, zd1, ones1)
    hq = _dense_layer(False, hq, mq, deg, W_self[1], W_neigh[1], b_l[1],
                      W_out, b_out)
    mq = _seg(src2, dst2, *hq, zq, zd1, ones1)
    oq = _dense_layer(True, hq, mq, deg, W_self[2], W_neigh[2], b_l[2],
                      W_out, b_out)

    tidx = jnp.concatenate([task_node_indices,
                            jnp.full((GT - T,), ag_node_idx, jnp.int32)])
    gq = _gather(*oq, tidx)
    fin = finished_task.astype(_f32).reshape(T, 1)
    return _readout(gq, fin)
